# baseline (speedup 1.0000x reference)
import functools
import math

import jax
import jax.numpy as jnp
from jax import lax
from jax.experimental import pallas as pl
from jax.experimental.pallas import tpu as pltpu
from jax.experimental.pallas import tpu_sc as plsc

N, E, D, G, P, S, L = 10000, 320000, 128, 2000, 32, 2, 3

NCORES, NSUB = 2, 16
NP = 10240
CH = 128
ROWS_PT = NP // NSUB
EPT = 9984
NCHF = EPT // CH
EMAIN = NCORES * NSUB * EPT
NEXTRA = (E - EMAIN) // CH
GCH = 64
GPW = NP // (NCORES * NSUB)
BN = 1024

_SDS = jax.ShapeDtypeStruct
_f32 = jnp.float32


def _sc_agg_body(h_hbm, ei_hbm, zrow_hbm, agg_out, *scr):
    eidx = scr[0:2]
    sidx = scr[2:4]
    didx = scr[4:6]
    rows = scr[6:8]
    agg_sh = scr[8]
    isem = scr[9:11]
    gsem = scr[11:13]

    c = lax.axis_index("c")
    s = lax.axis_index("s")
    w = c * NSUB + s
    r0 = pl.multiple_of(s * ROWS_PT, 8)
    pltpu.sync_copy(zrow_hbm.at[pl.ds(r0, ROWS_PT)], agg_sh.at[pl.ds(r0, ROWS_PT)])
    ebase = pl.multiple_of(w * EPT, 128)
    plsc.subcore_barrier()

    def eslice(j):
        b = pl.multiple_of(ebase + j * CH, 128)
        return ei_hbm.at[pl.ds(0, 2), pl.ds(b, CH)]

    def idx_start(j, t):
        pltpu.async_copy(eslice(j), eidx[t], isem[t])

    def idx_finish(j, t):
        pltpu.make_async_copy(eslice(j), eidx[t], isem[t]).wait()
        for k in range(CH // 16):
            sl = pl.ds(k * 16, 16)
            sidx[t][sl] = eidx[t][0, sl]
            didx[t][sl] = eidx[t][1, sl]
        pltpu.async_copy(h_hbm.at[sidx[t]], rows[t], gsem[t])

    def scatter(j, t):
        pltpu.make_async_copy(h_hbm.at[sidx[t]], rows[t], gsem[t]).wait()
        pltpu.sync_copy(rows[t], agg_sh.at[didx[t]], add=True)

    idx_start(0, 0)
    idx_start(1, 1)
    idx_finish(0, 0)

    def body(m, carry):
        scatter(2 * m, 0)

        @pl.when(2 * m + 2 < NCHF)
        def _():
            idx_start(2 * m + 2, 0)

        idx_finish(2 * m + 1, 1)
        scatter(2 * m + 1, 1)

        @pl.when(2 * m + 3 < NCHF)
        def _():
            idx_start(2 * m + 3, 1)

        @pl.when(2 * m + 2 < NCHF)
        def _():
            idx_finish(2 * m + 2, 0)
        return carry

    lax.fori_loop(0, NCHF // 2, body, 0)

    @pl.when(w < NEXTRA)
    def _():
        b = pl.multiple_of(EMAIN + w * CH, 128)
        pltpu.sync_copy(ei_hbm.at[pl.ds(0, 2), pl.ds(b, CH)], eidx[0])
        for k in range(CH // 16):
            sl = pl.ds(k * 16, 16)
            sidx[0][sl] = eidx[0][0, sl]
            didx[0][sl] = eidx[0][1, sl]
        pltpu.async_copy(h_hbm.at[sidx[0]], rows[0], gsem[0]).wait()
        pltpu.sync_copy(rows[0], agg_sh.at[didx[0]], add=True)

    plsc.subcore_barrier()
    pltpu.sync_copy(agg_sh.at[pl.ds(r0, ROWS_PT)], agg_out.at[c, pl.ds(r0, ROWS_PT)])


_sc_agg = pl.kernel(
    _sc_agg_body,
    out_type=_SDS((NCORES, NP, D), _f32),
    mesh=plsc.VectorSubcoreMesh(core_axis_name="c", subcore_axis_name="s",
                                num_cores=NCORES, num_subcores=NSUB),
    scratch_types=(
        [pltpu.VMEM((2, CH), jnp.int32)] * 2
        + [pltpu.VMEM((CH,), jnp.int32)] * 2
        + [pltpu.VMEM((CH,), jnp.int32)] * 2
        + [pltpu.VMEM((CH, D), _f32)] * 2
        + [pltpu.VMEM_SHARED((NP, D), _f32)]
        + [pltpu.SemaphoreType.DMA] * 4
    ),
)


def _sc_deg_body(ei_hbm, ones_hbm, zrow_hbm, deg_out, *scr):
    eidx = scr[0:2]
    didx = scr[2:4]
    ones_v, deg_sh = scr[4:6]
    isem = scr[6:8]

    c = lax.axis_index("c")
    s = lax.axis_index("s")
    w = c * NSUB + s
    r0 = pl.multiple_of(s * ROWS_PT, 8)
    pltpu.sync_copy(zrow_hbm.at[pl.ds(r0, ROWS_PT)], deg_sh.at[pl.ds(r0, ROWS_PT)])
    pltpu.sync_copy(ones_hbm, ones_v)
    ebase = pl.multiple_of(w * EPT, 128)
    plsc.subcore_barrier()

    def eslice(j):
        b = pl.multiple_of(ebase + j * CH, 128)
        return ei_hbm.at[pl.ds(0, 2), pl.ds(b, CH)]

    def idx_start(j, t):
        pltpu.async_copy(eslice(j), eidx[t], isem[t])

    def scatter(j, t):
        pltpu.make_async_copy(eslice(j), eidx[t], isem[t]).wait()
        for k in range(CH // 16):
            sl = pl.ds(k * 16, 16)
            didx[t][sl] = eidx[t][1, sl]
        pltpu.sync_copy(ones_v, deg_sh.at[didx[t]], add=True)

    idx_start(0, 0)

    def body(m, carry):
        idx_start(2 * m + 1, 1)
        scatter(2 * m, 0)

        @pl.when(m < NCHF // 2 - 1)
        def _():
            idx_start(2 * m + 2, 0)

        scatter(2 * m + 1, 1)
        return carry

    lax.fori_loop(0, NCHF // 2, body, 0)

    @pl.when(w < NEXTRA)
    def _():
        b = pl.multiple_of(EMAIN + w * CH, 128)
        pltpu.sync_copy(ei_hbm.at[pl.ds(0, 2), pl.ds(b, CH)], eidx[0])
        for k in range(CH // 16):
            sl = pl.ds(k * 16, 16)
            didx[0][sl] = eidx[0][1, sl]
        pltpu.sync_copy(ones_v, deg_sh.at[didx[0]], add=True)

    plsc.subcore_barrier()
    pltpu.sync_copy(deg_sh.at[pl.ds(r0, ROWS_PT)], deg_out.at[c, pl.ds(r0, ROWS_PT)])


_sc_deg = pl.kernel(
    _sc_deg_body,
    out_type=_SDS((NCORES, NP, D), _f32),
    mesh=plsc.VectorSubcoreMesh(core_axis_name="c", subcore_axis_name="s",
                                num_cores=NCORES, num_subcores=NSUB),
    scratch_types=(
        [pltpu.VMEM((2, CH), jnp.int32)] * 2
        + [pltpu.VMEM((CH,), jnp.int32)] * 2
        + [pltpu.VMEM((CH, D), _f32),
           pltpu.VMEM_SHARED((NP, D), _f32)]
        + [pltpu.SemaphoreType.DMA] * 2
    ),
)


def _sc_gath_body(emb_hbm, glpT_hbm, gid_hbm, emb_out, gev_out,
                  gidx, erows, grows, sem):
    c = lax.axis_index("c")
    s = lax.axis_index("s")
    w = s * NCORES + c
    base0 = w * GPW

    def body(j, carry):
        b = pl.multiple_of(base0 + j * GCH, 8)
        pltpu.sync_copy(gid_hbm.at[pl.ds(b, GCH)], gidx)
        pltpu.async_copy(emb_hbm.at[gidx], erows, sem).wait()
        pltpu.async_copy(glpT_hbm.at[gidx], grows, sem).wait()
        pltpu.sync_copy(erows, emb_out.at[pl.ds(b, GCH)])
        pltpu.sync_copy(grows, gev_out.at[pl.ds(b, GCH)])
        return carry

    lax.fori_loop(0, GPW // GCH, body, 0)


_sc_gath = pl.kernel(
    _sc_gath_body,
    out_type=(_SDS((NP, D), _f32), _SDS((NP, D), _f32)),
    mesh=plsc.VectorSubcoreMesh(core_axis_name="c", subcore_axis_name="s",
                                num_cores=NCORES, num_subcores=NSUB),
    scratch_types=[
        pltpu.VMEM((GCH,), jnp.int32),
        pltpu.VMEM((GCH, D), _f32),
        pltpu.VMEM((GCH, D), _f32),
        pltpu.SemaphoreType.DMA,
    ],
)


def _tc_prep_body(co_ref, cW1_ref, cb1_ref, cW2_ref, cb2_ref, pgT_ref,
                  cmlp_ref, glpT_ref):
    co = co_ref[...]
    t = (co[:, 0:1] * cW1_ref[0:1, :] + co[:, 1:2] * cW1_ref[1:2, :]
         + cb1_ref[...])
    t = jnp.maximum(t, 0.0)
    cmlp_ref[...] = (jnp.dot(t, cW2_ref[...], preferred_element_type=_f32)
                     + cb2_ref[...])
    pgT = pgT_ref[...]
    m = jnp.max(pgT, axis=0, keepdims=True)
    lse = jnp.log(jnp.sum(jnp.exp(pgT - m), axis=0, keepdims=True)) + m
    glpT_ref[...] = jnp.concatenate(
        [pgT - lse, jnp.zeros((G, D - P), _f32)], axis=1)


def _tc_add_body(a_ref, b_ref, o_ref):
    o_ref[...] = a_ref[...] + b_ref[...]


def _tc_mlp_body(h_ref, aggA_ref, aggB_ref, dA_ref, dB_ref,
                 W1_ref, b1_ref, W2_ref, b2_ref, out_ref):
    h = h_ref[...]
    deg = jnp.maximum(dA_ref[0][:, 0:1] + dB_ref[0][:, 0:1], 1.0)
    agg = (aggA_ref[0] + aggB_ref[0]) / deg
    W1 = W1_ref[...]
    x = (jnp.dot(h, W1[:D], preferred_element_type=_f32)
         + jnp.dot(agg, W1[D:], preferred_element_type=_f32) + b1_ref[...])
    x = jnp.maximum(x, 0.0)
    x = jnp.dot(x, W2_ref[...], preferred_element_type=_f32) + b2_ref[...]
    out_ref[...] = h + jnp.maximum(x, 0.0)


_LOG2PI = math.log(2.0 * math.pi)


def _tc_head_body(h_ref, gev_ref, co_ref, aW_ref, ab_ref, pgl_ref, pp_ref,
                  pcmT_ref, pclvT_ref,
                  logits_ref, q_ref, gp_ref, pred_ref, ce_ref, pclp_ref):
    pgl = pgl_ref[...]
    m = jnp.max(pgl, axis=1, keepdims=True)
    lse = jnp.log(jnp.sum(jnp.exp(pgl - m), axis=1, keepdims=True)) + m
    gp = jnp.exp(pgl - lse)
    gp_ref[...] = gp
    lv = jnp.clip(pclvT_ref[...], -4.0, 4.0)
    co = co_ref[...]
    ce = jnp.zeros((BN, P), _f32)
    for s_ in range(S):
        dlt = co[:, s_:s_ + 1] - pcmT_ref[s_:s_ + 1, :]
        ce = ce + dlt * dlt * jnp.exp(-lv[s_:s_ + 1, :]) + lv[s_:s_ + 1, :] + _LOG2PI
    ce = -0.5 * ce
    ce_ref[...] = ce
    logits = (jnp.dot(h_ref[...], aW_ref[...], preferred_element_type=_f32)
              + ab_ref[...] + gev_ref[...] + ce + pp_ref[...])
    logits_ref[...] = logits
    lm = jnp.max(logits, axis=1, keepdims=True)
    e2 = jnp.exp(logits - lm)
    q = e2 / jnp.sum(e2, axis=1, keepdims=True)
    q_ref[...] = q
    pred_ref[...] = jnp.dot(q, gp, preferred_element_type=_f32)
    t = jnp.log(jnp.maximum(q, 1e-9)) + ce
    tm = jnp.max(t, axis=1, keepdims=True)
    pclp_ref[...] = tm + jnp.log(jnp.sum(jnp.exp(t - tm), axis=1, keepdims=True))


def _row_spec(w):
    return pl.BlockSpec((BN, w), lambda i: (i, 0))


def _const_spec(shape):
    return pl.BlockSpec(shape, lambda i: tuple(0 for _ in shape))


_tc_prep = pl.pallas_call(
    _tc_prep_body,
    out_shape=(_SDS((NP, D), _f32), _SDS((G, D), _f32)),
)

_tc_add = pl.pallas_call(
    _tc_add_body,
    grid=(NP // BN,),
    in_specs=[_row_spec(D), _row_spec(D)],
    out_specs=_row_spec(D),
    out_shape=_SDS((NP, D), _f32),
)

_tc_mlp = pl.pallas_call(
    _tc_mlp_body,
    grid=(NP // BN,),
    in_specs=[_row_spec(D),
              pl.BlockSpec((1, BN, D), lambda i: (0, i, 0)),
              pl.BlockSpec((1, BN, D), lambda i: (1, i, 0)),
              pl.BlockSpec((1, BN, D), lambda i: (0, i, 0)),
              pl.BlockSpec((1, BN, D), lambda i: (1, i, 0)),
              _const_spec((2 * D, D)), _const_spec((1, D)),
              _const_spec((D, D)), _const_spec((1, D))],
    out_specs=_row_spec(D),
    out_shape=_SDS((NP, D), _f32),
)

_tc_head = pl.pallas_call(
    _tc_head_body,
    grid=(NP // BN,),
    in_specs=[_row_spec(D), _row_spec(P), _row_spec(S), _const_spec((D, P)),
              _const_spec((1, P)), _const_spec((P, G)), _const_spec((1, P)),
              _const_spec((S, P)), _const_spec((S, P))],
    out_specs=(_row_spec(P), _row_spec(P), _const_spec((P, G)), _row_spec(G),
               _row_spec(P), _row_spec(1)),
    out_shape=(_SDS((NP, P), _f32), _SDS((NP, P), _f32), _SDS((P, G), _f32),
               _SDS((NP, G), _f32), _SDS((NP, P), _f32), _SDS((NP, 1), _f32)),
)


def kernel(coords, gene_ids, edge_index, gene_emb, cW1, cb1, cW2, cb2,
           mpW1, mpb1, mpW2, mpb2, aW, ab, pg_logits, pp_logits,
           pc_means, pc_logvars):
    coords_p = jnp.pad(coords, ((0, NP - N), (0, 0)))
    gid_p = jnp.pad(gene_ids, (0, NP - N))
    zrow = jnp.zeros((NP, D), _f32)
    onesr = jnp.ones((CH, D), _f32)

    cmlp, glpT = _tc_prep(coords_p, cW1, cb1[None], cW2, cb2[None],
                          pg_logits.T)
    emb, gev = _sc_gath(gene_emb, glpT, gid_p)
    h = _tc_add(emb, cmlp)
    degP = _sc_deg(edge_index, onesr, zrow)
    for i in range(L):
        aggP = _sc_agg(h, edge_index, zrow)
        h = _tc_mlp(h, aggP, aggP, degP, degP,
                    mpW1[i], mpb1[i][None], mpW2[i], mpb2[i][None])
    logits, q, gp, pred, ce, pclp = _tc_head(
        h, gev[:, :P], coords_p, aW, ab[None], pg_logits, pp_logits[None],
        pc_means.T, pc_logvars.T)
    return (h[:N], logits[:N], q[:N], gp, pred[:N], ce[:N], pclp[:N, 0])


# speedup vs baseline: 1.7586x; 1.7586x over previous
import math

import jax
import jax.numpy as jnp
from jax import lax
from jax.experimental import pallas as pl
from jax.experimental.pallas import tpu as pltpu
from jax.experimental.pallas import tpu_sc as plsc

N, E, D, G, P, S, L = 10000, 320000, 128, 2000, 32, 2, 3

NCORES, NSUB = 2, 16
NP = 10240
CH = 128
ROWS_PT = NP // NSUB
EPC = E // NCORES
EPT = EPC // NSUB
NCHF = EPT // CH
TAIL = EPT - NCHF * CH
GCH = 64
GPW = NP // (NCORES * NSUB)
BN = 1000

_SDS = jax.ShapeDtypeStruct
_f32 = jnp.float32


def _sc_agg_body(h_hbm, ei_hbm, zrow_hbm,
                 agg_out,
                 sidx_all, didx0, rows0, didx1, rows1, rows_t, didx_t,
                 agg_sh, gsem0, gsem1, dsem0, dsem1):
    c = lax.axis_index("c")
    s = lax.axis_index("s")
    r0 = pl.multiple_of(s * ROWS_PT, 8)
    pltpu.sync_copy(zrow_hbm.at[pl.ds(r0, ROWS_PT)], agg_sh.at[pl.ds(r0, ROWS_PT)])
    ebase = pl.multiple_of((c * NSUB + s) * EPT, 8)
    pltpu.sync_copy(ei_hbm.at[pl.ds(ebase, EPT)], sidx_all)
    plsc.subcore_barrier()

    def start(j, didx, rows, gsem, dsem):
        off = pl.multiple_of(j * CH, 8)
        pltpu.async_copy(ei_hbm.at[pl.ds(E + ebase + off, CH)], didx, dsem)
        pltpu.async_copy(h_hbm.at[sidx_all.at[pl.ds(off, CH)]], rows, gsem)

    def finish(j, didx, rows, gsem, dsem):
        off = pl.multiple_of(j * CH, 8)
        pltpu.make_async_copy(
            ei_hbm.at[pl.ds(E + ebase + off, CH)], didx, dsem).wait()
        pltpu.make_async_copy(
            h_hbm.at[sidx_all.at[pl.ds(off, CH)]], rows, gsem).wait()
        pltpu.sync_copy(rows, agg_sh.at[didx], add=True)

    start(0, didx0, rows0, gsem0, dsem0)

    def body(m, carry):
        start(2 * m + 1, didx1, rows1, gsem1, dsem1)
        finish(2 * m, didx0, rows0, gsem0, dsem0)

        @pl.when(m < NCHF // 2 - 1)
        def _():
            start(2 * m + 2, didx0, rows0, gsem0, dsem0)

        finish(2 * m + 1, didx1, rows1, gsem1, dsem1)
        return carry

    lax.fori_loop(0, NCHF // 2, body, 0)
    toff = pl.multiple_of(NCHF * CH, 8)
    pltpu.sync_copy(ei_hbm.at[pl.ds(E + ebase + toff, TAIL)], didx_t)
    pltpu.async_copy(
        h_hbm.at[sidx_all.at[pl.ds(toff, TAIL)]], rows_t, gsem0).wait()
    pltpu.sync_copy(rows_t, agg_sh.at[didx_t], add=True)

    plsc.subcore_barrier()
    pltpu.sync_copy(agg_sh.at[pl.ds(r0, ROWS_PT)], agg_out.at[c, pl.ds(r0, ROWS_PT)])


_sc_agg = pl.kernel(
    _sc_agg_body,
    out_type=_SDS((NCORES, NP, D), _f32),
    mesh=plsc.VectorSubcoreMesh(core_axis_name="c", subcore_axis_name="s",
                                num_cores=NCORES, num_subcores=NSUB),
    scratch_types=[
        pltpu.VMEM((EPT,), jnp.int32),
        pltpu.VMEM((CH,), jnp.int32),
        pltpu.VMEM((CH, D), _f32),
        pltpu.VMEM((CH,), jnp.int32),
        pltpu.VMEM((CH, D), _f32),
        pltpu.VMEM((TAIL, D), _f32),
        pltpu.VMEM((TAIL,), jnp.int32),
        pltpu.VMEM_SHARED((NP, D), _f32),
        pltpu.SemaphoreType.DMA,
        pltpu.SemaphoreType.DMA,
        pltpu.SemaphoreType.DMA,
        pltpu.SemaphoreType.DMA,
    ],
)


def _sc_deg_body(ei_hbm, ones_hbm, zrow_hbm,
                 deg_out,
                 didx0, didx1, didx_t, ones_v, deg_sh, dsem0, dsem1):
    c = lax.axis_index("c")
    s = lax.axis_index("s")
    r0 = pl.multiple_of(s * ROWS_PT, 8)
    pltpu.sync_copy(zrow_hbm.at[pl.ds(r0, ROWS_PT)], deg_sh.at[pl.ds(r0, ROWS_PT)])
    pltpu.sync_copy(ones_hbm, ones_v)
    ebase = pl.multiple_of((c * NSUB + s) * EPT, 8)
    plsc.subcore_barrier()

    def start(j, didx, dsem):
        off = pl.multiple_of(j * CH, 8)
        pltpu.async_copy(ei_hbm.at[pl.ds(E + ebase + off, CH)], didx, dsem)

    def finish(j, didx, dsem):
        off = pl.multiple_of(j * CH, 8)
        pltpu.make_async_copy(
            ei_hbm.at[pl.ds(E + ebase + off, CH)], didx, dsem).wait()
        pltpu.sync_copy(ones_v, deg_sh.at[didx], add=True)

    start(0, didx0, dsem0)

    def body(m, carry):
        start(2 * m + 1, didx1, dsem1)
        finish(2 * m, didx0, dsem0)

        @pl.when(m < NCHF // 2 - 1)
        def _():
            start(2 * m + 2, didx0, dsem0)

        finish(2 * m + 1, didx1, dsem1)
        return carry

    lax.fori_loop(0, NCHF // 2, body, 0)
    toff = pl.multiple_of(NCHF * CH, 8)
    pltpu.sync_copy(ei_hbm.at[pl.ds(E + ebase + toff, TAIL)], didx_t)
    pltpu.sync_copy(ones_v.at[pl.ds(0, TAIL)], deg_sh.at[didx_t], add=True)

    plsc.subcore_barrier()
    pltpu.sync_copy(deg_sh.at[pl.ds(r0, ROWS_PT)], deg_out.at[c, pl.ds(r0, ROWS_PT)])


_sc_deg = pl.kernel(
    _sc_deg_body,
    out_type=_SDS((NCORES, NP, D), _f32),
    mesh=plsc.VectorSubcoreMesh(core_axis_name="c", subcore_axis_name="s",
                                num_cores=NCORES, num_subcores=NSUB),
    scratch_types=[
        pltpu.VMEM((CH,), jnp.int32),
        pltpu.VMEM((CH,), jnp.int32),
        pltpu.VMEM((TAIL,), jnp.int32),
        pltpu.VMEM((CH, D), _f32),
        pltpu.VMEM_SHARED((NP, D), _f32),
        pltpu.SemaphoreType.DMA,
        pltpu.SemaphoreType.DMA,
    ],
)


def _sc_gath_body(emb_hbm, glpT_hbm, gid_hbm, emb_out, gev_out,
                  gidx, erows, grows, sem):
    c = lax.axis_index("c")
    s = lax.axis_index("s")
    w = s * NCORES + c
    base0 = w * GPW

    def body(j, carry):
        b = pl.multiple_of(base0 + j * GCH, 8)
        pltpu.sync_copy(gid_hbm.at[pl.ds(b, GCH)], gidx)
        pltpu.async_copy(emb_hbm.at[gidx], erows, sem).wait()
        pltpu.async_copy(glpT_hbm.at[gidx], grows, sem).wait()
        pltpu.sync_copy(erows, emb_out.at[pl.ds(b, GCH)])
        pltpu.sync_copy(grows, gev_out.at[pl.ds(b, GCH)])
        return carry

    lax.fori_loop(0, GPW // GCH, body, 0)


_sc_gath = pl.kernel(
    _sc_gath_body,
    out_type=(_SDS((NP, D), _f32), _SDS((NP, D), _f32)),
    mesh=plsc.VectorSubcoreMesh(core_axis_name="c", subcore_axis_name="s",
                                num_cores=NCORES, num_subcores=NSUB),
    scratch_types=[
        pltpu.VMEM((GCH,), jnp.int32),
        pltpu.VMEM((GCH, D), _f32),
        pltpu.VMEM((GCH, D), _f32),
        pltpu.SemaphoreType.DMA,
    ],
)


def _tc_prep_body(co_ref, cW1_ref, cb1_ref, cW2_ref, cb2_ref, pgT_ref,
                  cmlp_ref, glpT_ref):
    co = co_ref[...]
    t = (co[:, 0:1] * cW1_ref[0:1, :] + co[:, 1:2] * cW1_ref[1:2, :]
         + cb1_ref[...])
    t = jnp.maximum(t, 0.0)
    cmlp_ref[...] = (jnp.dot(t, cW2_ref[...], preferred_element_type=_f32)
                     + cb2_ref[...])
    pgT = pgT_ref[...]
    m = jnp.max(pgT, axis=0, keepdims=True)
    lse = jnp.log(jnp.sum(jnp.exp(pgT - m), axis=0, keepdims=True)) + m
    glpT_ref[...] = jnp.concatenate(
        [pgT - lse, jnp.zeros((G, D - P), _f32)], axis=1)


def _tc_add_body(a_ref, b_ref, o_ref):
    o_ref[...] = a_ref[...] + b_ref[...]


def _tc_mlp_body(h_ref, aggA_ref, aggB_ref, dA_ref, dB_ref,
                 W1_ref, b1_ref, W2_ref, b2_ref, out_ref):
    h = h_ref[...]
    deg = jnp.maximum(dA_ref[0][:, 0:1] + dB_ref[0][:, 0:1], 1.0)
    agg = (aggA_ref[0] + aggB_ref[0]) / deg
    W1 = W1_ref[...]
    x = (jnp.dot(h, W1[:D], preferred_element_type=_f32)
         + jnp.dot(agg, W1[D:], preferred_element_type=_f32) + b1_ref[...])
    x = jnp.maximum(x, 0.0)
    x = jnp.dot(x, W2_ref[...], preferred_element_type=_f32) + b2_ref[...]
    out_ref[...] = h + jnp.maximum(x, 0.0)


_LOG2PI = math.log(2.0 * math.pi)


def _tc_head_body(h_ref, gev_ref, co_ref, aW_ref, ab_ref, pgl_ref, pp_ref,
                  pcmT_ref, pclvT_ref,
                  logits_ref, q_ref, gp_ref, pred_ref, ce_ref, pclp_ref):
    pgl = pgl_ref[...]
    m = jnp.max(pgl, axis=1, keepdims=True)
    lse = jnp.log(jnp.sum(jnp.exp(pgl - m), axis=1, keepdims=True)) + m
    gp = jnp.exp(pgl - lse)
    gp_ref[...] = gp
    lv = jnp.clip(pclvT_ref[...], -4.0, 4.0)
    co = co_ref[...]
    ce = jnp.zeros((BN, P), _f32)
    for s_ in range(S):
        dlt = co[:, s_:s_ + 1] - pcmT_ref[s_:s_ + 1, :]
        ce = ce + dlt * dlt * jnp.exp(-lv[s_:s_ + 1, :]) + lv[s_:s_ + 1, :] + _LOG2PI
    ce = -0.5 * ce
    ce_ref[...] = ce
    logits = (jnp.dot(h_ref[...], aW_ref[...], preferred_element_type=_f32)
              + ab_ref[...] + gev_ref[...][:, :P] + ce + pp_ref[...])
    logits_ref[...] = logits
    lm = jnp.max(logits, axis=1, keepdims=True)
    e2 = jnp.exp(logits - lm)
    q = e2 / jnp.sum(e2, axis=1, keepdims=True)
    q_ref[...] = q
    pred_ref[...] = jnp.dot(q, gp, preferred_element_type=_f32)
    t = jnp.log(jnp.maximum(q, 1e-9)) + ce
    tm = jnp.max(t, axis=1, keepdims=True)
    pclp_ref[...] = tm + jnp.log(jnp.sum(jnp.exp(t - tm), axis=1, keepdims=True))


def _row_spec(w):
    return pl.BlockSpec((BN, w), lambda i: (i, 0))


def _part_spec(cidx):
    return pl.BlockSpec((1, BN, D), lambda i, _c=cidx: (_c, i, 0))


def _const_spec(shape):
    return pl.BlockSpec(shape, lambda i: tuple(0 for _ in shape))


_tc_prep = pl.pallas_call(
    _tc_prep_body,
    out_shape=(_SDS((N, D), _f32), _SDS((G, D), _f32)),
)

_tc_add = pl.pallas_call(
    _tc_add_body,
    grid=(N // BN,),
    in_specs=[_row_spec(D), _row_spec(D)],
    out_specs=_row_spec(D),
    out_shape=_SDS((N, D), _f32),
)

_tc_mlp = pl.pallas_call(
    _tc_mlp_body,
    grid=(N // BN,),
    in_specs=[_row_spec(D), _part_spec(0), _part_spec(1), _part_spec(0),
              _part_spec(1), _const_spec((2 * D, D)), _const_spec((1, D)),
              _const_spec((D, D)), _const_spec((1, D))],
    out_specs=_row_spec(D),
    out_shape=_SDS((N, D), _f32),
)

_tc_head = pl.pallas_call(
    _tc_head_body,
    grid=(N // BN,),
    in_specs=[_row_spec(D), _row_spec(D), _row_spec(S), _const_spec((D, P)),
              _const_spec((1, P)), _const_spec((P, G)), _const_spec((1, P)),
              _const_spec((S, P)), _const_spec((S, P))],
    out_specs=(_row_spec(P), _row_spec(P), _const_spec((P, G)), _row_spec(G),
               _row_spec(P), _row_spec(1)),
    out_shape=(_SDS((N, P), _f32), _SDS((N, P), _f32), _SDS((P, G), _f32),
               _SDS((N, G), _f32), _SDS((N, P), _f32), _SDS((N, 1), _f32)),
)


def kernel(coords, gene_ids, edge_index, gene_emb, cW1, cb1, cW2, cb2,
           mpW1, mpb1, mpW2, mpb2, aW, ab, pg_logits, pp_logits,
           pc_means, pc_logvars):
    gid_p = jnp.pad(gene_ids, (0, NP - N))
    ei_flat = edge_index.reshape(2 * E)
    zrow = jnp.zeros((NP, D), _f32)
    onesr = jnp.ones((CH, D), _f32)

    cmlp, glpT = _tc_prep(coords, cW1, cb1[None], cW2, cb2[None],
                          pg_logits.T)
    emb, gev = _sc_gath(gene_emb, glpT, gid_p)
    h = _tc_add(emb, cmlp)
    degP = _sc_deg(ei_flat, onesr, zrow)
    for i in range(L):
        aggP = _sc_agg(h, ei_flat, zrow)
        h = _tc_mlp(h, aggP, aggP, degP, degP,
                    mpW1[i], mpb1[i][None], mpW2[i], mpb2[i][None])
    logits, q, gp, pred, ce, pclp = _tc_head(
        h, gev, coords, aW, ab[None], pg_logits, pp_logits[None],
        pc_means.T, pc_logvars.T)
    return (h, logits, q, gp, pred, ce, pclp.reshape(N))


# speedup vs baseline: 1.7704x; 1.0067x over previous
import math

import jax
import jax.numpy as jnp
from jax import lax
from jax.experimental import pallas as pl
from jax.experimental.pallas import tpu as pltpu
from jax.experimental.pallas import tpu_sc as plsc

N, E, D, G, P, S, L = 10000, 320000, 128, 2000, 32, 2, 3

NCORES, NSUB = 2, 16
NP = 10240
CH = 128
ROWS_PT = NP // NSUB
EPC = E // NCORES
EPT = EPC // NSUB
NCHF = EPT // CH
TAIL = EPT - NCHF * CH
GCH = 64
GPW = NP // (NCORES * NSUB)
BN = 1000

_SDS = jax.ShapeDtypeStruct
_f32 = jnp.float32


ACH = 64
ANCH = EPT // ACH
ATAIL = EPT - ANCH * ACH


def _sc_agg_body(h_hbm, ei_hbm, zrow_hbm, agg_out, *scr):
    sidx_all = scr[0]
    didx = scr[1:5]
    rows = scr[5:9]
    rows_t, didx_t, agg_sh = scr[9:12]
    gsem = scr[12:16]
    dsem = scr[16:20]
    ssem = scr[20:24]

    c = lax.axis_index("c")
    s = lax.axis_index("s")
    r0 = pl.multiple_of(s * ROWS_PT, 8)
    pltpu.sync_copy(zrow_hbm.at[pl.ds(r0, ROWS_PT)], agg_sh.at[pl.ds(r0, ROWS_PT)])
    ebase = pl.multiple_of((c * NSUB + s) * EPT, 8)
    pltpu.sync_copy(ei_hbm.at[pl.ds(ebase, EPT)], sidx_all)
    plsc.subcore_barrier()

    def start(j, t):
        off = pl.multiple_of(j * ACH, 8)
        pltpu.async_copy(ei_hbm.at[pl.ds(E + ebase + off, ACH)], didx[t], dsem[t])
        pltpu.async_copy(h_hbm.at[sidx_all.at[pl.ds(off, ACH)]], rows[t], gsem[t])

    def wait_scatter(t):
        pltpu.make_async_copy(rows[t], agg_sh.at[didx[t]], ssem[t]).wait()

    def process(j, t):
        off = pl.multiple_of(j * ACH, 8)
        pltpu.make_async_copy(
            ei_hbm.at[pl.ds(E + ebase + off, ACH)], didx[t], dsem[t]).wait()
        pltpu.make_async_copy(
            h_hbm.at[sidx_all.at[pl.ds(off, ACH)]], rows[t], gsem[t]).wait()
        pltpu.async_copy(rows[t], agg_sh.at[didx[t]], ssem[t], add=True)

    for t in range(4):
        start(t, t)

    def body(m, carry):
        process(4 * m + 0, 0)
        process(4 * m + 1, 1)
        wait_scatter(0)
        start(4 * m + 4, 0)
        process(4 * m + 2, 2)
        wait_scatter(1)
        start(4 * m + 5, 1)
        process(4 * m + 3, 3)
        wait_scatter(2)
        start(4 * m + 6, 2)
        wait_scatter(3)
        start(4 * m + 7, 3)
        return carry

    lax.fori_loop(0, ANCH // 4 - 1, body, 0)
    base = ANCH - 4
    process(base + 0, 0)
    process(base + 1, 1)
    wait_scatter(0)
    process(base + 2, 2)
    wait_scatter(1)
    process(base + 3, 3)
    wait_scatter(2)
    wait_scatter(3)

    toff = pl.multiple_of(ANCH * ACH, 8)
    pltpu.sync_copy(ei_hbm.at[pl.ds(E + ebase + toff, ATAIL)], didx_t)
    pltpu.async_copy(
        h_hbm.at[sidx_all.at[pl.ds(toff, ATAIL)]], rows_t, gsem[0]).wait()
    pltpu.sync_copy(rows_t, agg_sh.at[didx_t], add=True)

    plsc.subcore_barrier()
    pltpu.sync_copy(agg_sh.at[pl.ds(r0, ROWS_PT)], agg_out.at[c, pl.ds(r0, ROWS_PT)])


_sc_agg = pl.kernel(
    _sc_agg_body,
    out_type=_SDS((NCORES, NP, D), _f32),
    mesh=plsc.VectorSubcoreMesh(core_axis_name="c", subcore_axis_name="s",
                                num_cores=NCORES, num_subcores=NSUB),
    scratch_types=(
        [pltpu.VMEM((EPT,), jnp.int32)]
        + [pltpu.VMEM((ACH,), jnp.int32)] * 4
        + [pltpu.VMEM((ACH, D), _f32)] * 4
        + [pltpu.VMEM((TAIL, D), _f32),
           pltpu.VMEM((TAIL,), jnp.int32),
           pltpu.VMEM_SHARED((NP, D), _f32)]
        + [pltpu.SemaphoreType.DMA] * 12
    ),
)


def _sc_deg_body(ei_hbm, ones_hbm, zrow_hbm,
                 deg_out,
                 didx0, didx1, didx_t, ones_v, deg_sh, dsem0, dsem1):
    c = lax.axis_index("c")
    s = lax.axis_index("s")
    r0 = pl.multiple_of(s * ROWS_PT, 8)
    pltpu.sync_copy(zrow_hbm.at[pl.ds(r0, ROWS_PT)], deg_sh.at[pl.ds(r0, ROWS_PT)])
    pltpu.sync_copy(ones_hbm, ones_v)
    ebase = pl.multiple_of((c * NSUB + s) * EPT, 8)
    plsc.subcore_barrier()

    def start(j, didx, dsem):
        off = pl.multiple_of(j * CH, 8)
        pltpu.async_copy(ei_hbm.at[pl.ds(E + ebase + off, CH)], didx, dsem)

    def finish(j, didx, dsem):
        off = pl.multiple_of(j * CH, 8)
        pltpu.make_async_copy(
            ei_hbm.at[pl.ds(E + ebase + off, CH)], didx, dsem).wait()
        pltpu.sync_copy(ones_v, deg_sh.at[didx], add=True)

    start(0, didx0, dsem0)

    def body(m, carry):
        start(2 * m + 1, didx1, dsem1)
        finish(2 * m, didx0, dsem0)

        @pl.when(m < NCHF // 2 - 1)
        def _():
            start(2 * m + 2, didx0, dsem0)

        finish(2 * m + 1, didx1, dsem1)
        return carry

    lax.fori_loop(0, NCHF // 2, body, 0)
    toff = pl.multiple_of(NCHF * CH, 8)
    pltpu.sync_copy(ei_hbm.at[pl.ds(E + ebase + toff, TAIL)], didx_t)
    pltpu.sync_copy(ones_v.at[pl.ds(0, TAIL)], deg_sh.at[didx_t], add=True)

    plsc.subcore_barrier()
    pltpu.sync_copy(deg_sh.at[pl.ds(r0, ROWS_PT)], deg_out.at[c, pl.ds(r0, ROWS_PT)])


_sc_deg = pl.kernel(
    _sc_deg_body,
    out_type=_SDS((NCORES, NP, D), _f32),
    mesh=plsc.VectorSubcoreMesh(core_axis_name="c", subcore_axis_name="s",
                                num_cores=NCORES, num_subcores=NSUB),
    scratch_types=[
        pltpu.VMEM((CH,), jnp.int32),
        pltpu.VMEM((CH,), jnp.int32),
        pltpu.VMEM((TAIL,), jnp.int32),
        pltpu.VMEM((CH, D), _f32),
        pltpu.VMEM_SHARED((NP, D), _f32),
        pltpu.SemaphoreType.DMA,
        pltpu.SemaphoreType.DMA,
    ],
)


def _sc_gath_body(emb_hbm, glpT_hbm, gid_hbm, emb_out, gev_out,
                  gidx, erows, grows, sem):
    c = lax.axis_index("c")
    s = lax.axis_index("s")
    w = s * NCORES + c
    base0 = w * GPW

    def body(j, carry):
        b = pl.multiple_of(base0 + j * GCH, 8)
        pltpu.sync_copy(gid_hbm.at[pl.ds(b, GCH)], gidx)
        pltpu.async_copy(emb_hbm.at[gidx], erows, sem).wait()
        pltpu.async_copy(glpT_hbm.at[gidx], grows, sem).wait()
        pltpu.sync_copy(erows, emb_out.at[pl.ds(b, GCH)])
        pltpu.sync_copy(grows, gev_out.at[pl.ds(b, GCH)])
        return carry

    lax.fori_loop(0, GPW // GCH, body, 0)


_sc_gath = pl.kernel(
    _sc_gath_body,
    out_type=(_SDS((NP, D), _f32), _SDS((NP, D), _f32)),
    mesh=plsc.VectorSubcoreMesh(core_axis_name="c", subcore_axis_name="s",
                                num_cores=NCORES, num_subcores=NSUB),
    scratch_types=[
        pltpu.VMEM((GCH,), jnp.int32),
        pltpu.VMEM((GCH, D), _f32),
        pltpu.VMEM((GCH, D), _f32),
        pltpu.SemaphoreType.DMA,
    ],
)


def _tc_prep_body(co_ref, cW1_ref, cb1_ref, cW2_ref, cb2_ref, pgT_ref,
                  cmlp_ref, glpT_ref):
    co = co_ref[...]
    t = (co[:, 0:1] * cW1_ref[0:1, :] + co[:, 1:2] * cW1_ref[1:2, :]
         + cb1_ref[...])
    t = jnp.maximum(t, 0.0)
    cmlp_ref[...] = (jnp.dot(t, cW2_ref[...], preferred_element_type=_f32)
                     + cb2_ref[...])
    pgT = pgT_ref[...]
    m = jnp.max(pgT, axis=0, keepdims=True)
    lse = jnp.log(jnp.sum(jnp.exp(pgT - m), axis=0, keepdims=True)) + m
    glpT_ref[...] = jnp.concatenate(
        [pgT - lse, jnp.zeros((G, D - P), _f32)], axis=1)


def _tc_add_body(a_ref, b_ref, o_ref):
    o_ref[...] = a_ref[...] + b_ref[...]


def _tc_mlp_body(h_ref, aggA_ref, aggB_ref, dA_ref, dB_ref,
                 W1_ref, b1_ref, W2_ref, b2_ref, out_ref):
    h = h_ref[...]
    deg = jnp.maximum(dA_ref[0][:, 0:1] + dB_ref[0][:, 0:1], 1.0)
    agg = (aggA_ref[0] + aggB_ref[0]) / deg
    W1 = W1_ref[...]
    x = (jnp.dot(h, W1[:D], preferred_element_type=_f32)
         + jnp.dot(agg, W1[D:], preferred_element_type=_f32) + b1_ref[...])
    x = jnp.maximum(x, 0.0)
    x = jnp.dot(x, W2_ref[...], preferred_element_type=_f32) + b2_ref[...]
    out_ref[...] = h + jnp.maximum(x, 0.0)


_LOG2PI = math.log(2.0 * math.pi)


def _tc_head_body(h_ref, gev_ref, co_ref, aW_ref, ab_ref, pgl_ref, pp_ref,
                  pcmT_ref, pclvT_ref,
                  logits_ref, q_ref, gp_ref, pred_ref, ce_ref, pclp_ref):
    pgl = pgl_ref[...]
    m = jnp.max(pgl, axis=1, keepdims=True)
    lse = jnp.log(jnp.sum(jnp.exp(pgl - m), axis=1, keepdims=True)) + m
    gp = jnp.exp(pgl - lse)
    gp_ref[...] = gp
    lv = jnp.clip(pclvT_ref[...], -4.0, 4.0)
    co = co_ref[...]
    ce = jnp.zeros((BN, P), _f32)
    for s_ in range(S):
        dlt = co[:, s_:s_ + 1] - pcmT_ref[s_:s_ + 1, :]
        ce = ce + dlt * dlt * jnp.exp(-lv[s_:s_ + 1, :]) + lv[s_:s_ + 1, :] + _LOG2PI
    ce = -0.5 * ce
    ce_ref[...] = ce
    logits = (jnp.dot(h_ref[...], aW_ref[...], preferred_element_type=_f32)
              + ab_ref[...] + gev_ref[...][:, :P] + ce + pp_ref[...])
    logits_ref[...] = logits
    lm = jnp.max(logits, axis=1, keepdims=True)
    e2 = jnp.exp(logits - lm)
    q = e2 / jnp.sum(e2, axis=1, keepdims=True)
    q_ref[...] = q
    pred_ref[...] = jnp.dot(q, gp, preferred_element_type=_f32)
    t = jnp.log(jnp.maximum(q, 1e-9)) + ce
    tm = jnp.max(t, axis=1, keepdims=True)
    pclp_ref[...] = tm + jnp.log(jnp.sum(jnp.exp(t - tm), axis=1, keepdims=True))


def _row_spec(w):
    return pl.BlockSpec((BN, w), lambda i: (i, 0))


def _part_spec(cidx):
    return pl.BlockSpec((1, BN, D), lambda i, _c=cidx: (_c, i, 0))


def _const_spec(shape):
    return pl.BlockSpec(shape, lambda i: tuple(0 for _ in shape))


_tc_prep = pl.pallas_call(
    _tc_prep_body,
    out_shape=(_SDS((N, D), _f32), _SDS((G, D), _f32)),
)

_tc_add = pl.pallas_call(
    _tc_add_body,
    grid=(N // BN,),
    in_specs=[_row_spec(D), _row_spec(D)],
    out_specs=_row_spec(D),
    out_shape=_SDS((N, D), _f32),
)

_tc_mlp = pl.pallas_call(
    _tc_mlp_body,
    grid=(N // BN,),
    in_specs=[_row_spec(D), _part_spec(0), _part_spec(1), _part_spec(0),
              _part_spec(1), _const_spec((2 * D, D)), _const_spec((1, D)),
              _const_spec((D, D)), _const_spec((1, D))],
    out_specs=_row_spec(D),
    out_shape=_SDS((N, D), _f32),
)

_tc_head = pl.pallas_call(
    _tc_head_body,
    grid=(N // BN,),
    in_specs=[_row_spec(D), _row_spec(D), _row_spec(S), _const_spec((D, P)),
              _const_spec((1, P)), _const_spec((P, G)), _const_spec((1, P)),
              _const_spec((S, P)), _const_spec((S, P))],
    out_specs=(_row_spec(P), _row_spec(P), _const_spec((P, G)), _row_spec(G),
               _row_spec(P), _row_spec(1)),
    out_shape=(_SDS((N, P), _f32), _SDS((N, P), _f32), _SDS((P, G), _f32),
               _SDS((N, G), _f32), _SDS((N, P), _f32), _SDS((N, 1), _f32)),
)


def kernel(coords, gene_ids, edge_index, gene_emb, cW1, cb1, cW2, cb2,
           mpW1, mpb1, mpW2, mpb2, aW, ab, pg_logits, pp_logits,
           pc_means, pc_logvars):
    gid_p = jnp.pad(gene_ids, (0, NP - N))
    ei_flat = edge_index.reshape(2 * E)
    zrow = jnp.zeros((NP, D), _f32)
    onesr = jnp.ones((CH, D), _f32)

    cmlp, glpT = _tc_prep(coords, cW1, cb1[None], cW2, cb2[None],
                          pg_logits.T)
    emb, gev = _sc_gath(gene_emb, glpT, gid_p)
    h = _tc_add(emb, cmlp)
    degP = _sc_deg(ei_flat, onesr, zrow)
    for i in range(L):
        aggP = _sc_agg(h, ei_flat, zrow)
        h = _tc_mlp(h, aggP, aggP, degP, degP,
                    mpW1[i], mpb1[i][None], mpW2[i], mpb2[i][None])
    logits, q, gp, pred, ce, pclp = _tc_head(
        h, gev, coords, aW, ab[None], pg_logits, pp_logits[None],
        pc_means.T, pc_logvars.T)
    return (h, logits, q, gp, pred, ce, pclp.reshape(N))
